# baseline (speedup 1.0000x reference)
import functools

import jax
import jax.numpy as jnp
from jax import lax
from jax.experimental import pallas as pl
from jax.experimental.pallas import tpu as pltpu
from jax.experimental.pallas import tpu_sc as plsc

SEQ_BLK = 2048

NUM_WORKERS = 32
ROW_CHUNK = 64


def _add_kernel(x_ref, e_ref, o_ref):
    o_ref[0] = x_ref[0] + e_ref[...]


def _kernel_tc(x, emb_table):
    B, S, D = x.shape
    grid = (S // SEQ_BLK, B)
    return pl.pallas_call(
        _add_kernel,
        grid=grid,
        in_specs=[
            pl.BlockSpec((1, SEQ_BLK, D), lambda s, b: (b, s, 0)),
            pl.BlockSpec((SEQ_BLK, D), lambda s, b: (s, 0)),
        ],
        out_specs=pl.BlockSpec((1, SEQ_BLK, D), lambda s, b: (b, s, 0)),
        out_shape=jax.ShapeDtypeStruct((B, S, D), x.dtype),
    )(x, emb_table)


def _kernel_sc(x, emb_table, n_rows=None):
    B, S, D = x.shape
    N = B * S if n_rows is None else n_rows
    rows_per_worker = N // NUM_WORKERS
    n_chunks = rows_per_worker // ROW_CHUNK
    xf = x.reshape(B * S, D)
    pos = jnp.arange(S, dtype=jnp.int32)
    mesh = plsc.VectorSubcoreMesh(core_axis_name="c", subcore_axis_name="s")

    @functools.partial(
        pl.kernel,
        mesh=mesh,
        out_type=jax.ShapeDtypeStruct((N, D), jnp.float32),
        scratch_types=[
            pltpu.VMEM((ROW_CHUNK, D), jnp.float32),
            pltpu.VMEM((ROW_CHUNK,), jnp.int32),
            pltpu.SemaphoreType.DMA,
        ],
    )
    def sc_body(xf_hbm, pos_hbm, emb_hbm, out_hbm, xv, idx_v, sem):
        wid = lax.axis_index("c") * 16 + lax.axis_index("s")
        base = wid * rows_per_worker

        def chunk(c, carry):
            row0 = base + c * ROW_CHUNK
            s0 = lax.rem(row0, S)
            pltpu.sync_copy(xf_hbm.at[pl.ds(row0, ROW_CHUNK), :], xv)
            pltpu.sync_copy(pos_hbm.at[pl.ds(s0, ROW_CHUNK)], idx_v)
            pltpu.async_copy(emb_hbm.at[idx_v], xv, sem, add=True).wait()
            pltpu.sync_copy(xv, out_hbm.at[pl.ds(row0, ROW_CHUNK), :])
            return carry

        lax.fori_loop(0, n_chunks, chunk, 0)

    out = sc_body(xf, pos, emb_table)
    return out


def kernel(x, emb_table):
    if x.ndim == 2:
        return kernel(x[None], emb_table)[0]
    return _kernel_sc(x, emb_table).reshape(x.shape)


# speedup vs baseline: 2.1704x; 2.1704x over previous
import jax
import jax.numpy as jnp
from jax.experimental import pallas as pl
from jax.experimental.pallas import tpu as pltpu

SEQ_BLK = 2048


def _add_kernel(x_ref, e_ref, o_ref):
    o_ref[0] = x_ref[0] + e_ref[...]


def _kernel_tc(x, emb_table):
    B, S, D = x.shape
    grid = (S // SEQ_BLK, B)
    return pl.pallas_call(
        _add_kernel,
        grid=grid,
        in_specs=[
            pl.BlockSpec((1, SEQ_BLK, D), lambda s, b: (b, s, 0)),
            pl.BlockSpec((SEQ_BLK, D), lambda s, b: (s, 0)),
        ],
        out_specs=pl.BlockSpec((1, SEQ_BLK, D), lambda s, b: (b, s, 0)),
        out_shape=jax.ShapeDtypeStruct((B, S, D), x.dtype),
        compiler_params=pltpu.CompilerParams(
            vmem_limit_bytes=100 * 1024 * 1024,
        ),
    )(x, emb_table)


def kernel(x, emb_table):
    if x.ndim == 2:
        return kernel(x[None], emb_table)[0]
    return _kernel_tc(x, emb_table)


# speedup vs baseline: 2.1718x; 1.0007x over previous
import jax
import jax.numpy as jnp
from jax.experimental import pallas as pl
from jax.experimental.pallas import tpu as pltpu

SEQ_BLK = 4096
D_BLK = 512


def _add_kernel(x_ref, e_ref, o_ref):
    o_ref[0] = x_ref[0] + e_ref[...]


def _kernel_tc(x, emb_table):
    B, S, D = x.shape
    grid = (D // D_BLK, B)
    return pl.pallas_call(
        _add_kernel,
        grid=grid,
        in_specs=[
            pl.BlockSpec((1, SEQ_BLK, D_BLK), lambda d, b: (b, 0, d)),
            pl.BlockSpec((SEQ_BLK, D_BLK), lambda d, b: (0, d)),
        ],
        out_specs=pl.BlockSpec((1, SEQ_BLK, D_BLK), lambda d, b: (b, 0, d)),
        out_shape=jax.ShapeDtypeStruct((B, S, D), x.dtype),
        compiler_params=pltpu.CompilerParams(
            vmem_limit_bytes=100 * 1024 * 1024,
        ),
    )(x, emb_table)


def kernel(x, emb_table):
    if x.ndim == 2:
        return kernel(x[None], emb_table)[0]
    return _kernel_tc(x, emb_table)
